# Initial kernel scaffold; baseline (speedup 1.0000x reference)
import jax
import jax.numpy as jnp
from jax.experimental import pallas as pl


def kernel(preds, targets_list):
    raise NotImplementedError("write your pallas kernel here")


# speedup vs baseline: 3.9324x; 3.9324x over previous
import functools

import jax
import jax.numpy as jnp
from jax import lax
from jax.experimental import pallas as pl
from jax.experimental.pallas import tpu as pltpu
from jax.experimental.pallas import tpu_sc as plsc

B, C, H, W = 16, 8, 512, 512
HW = H * W
N = 64
NC, NS, L = 2, 16, 16
NW = NC * NS
TPW = (B * N) // NW


def _tc_softplus_sum(preds):
    def body(p_ref, o_ref):
        @pl.when(pl.program_id(0) == 0)
        def _():
            o_ref[0, 0] = 0.0

        x = p_ref[0, 0]
        o_ref[0, 0] += jnp.sum(
            jnp.maximum(x, 0.0) + jnp.log1p(jnp.exp(-jnp.abs(x))))

    return pl.pallas_call(
        body,
        grid=(B,),
        in_specs=[pl.BlockSpec((1, 1, H, W), lambda i: (i, 0, 0, 0))],
        out_specs=pl.BlockSpec(memory_space=pltpu.SMEM),
        out_shape=jax.ShapeDtypeStruct((1, 1), jnp.float32),
    )(preds)


def _sc_sparse_partials(preds_flat, tgt_flat):
    mesh = plsc.VectorSubcoreMesh(
        core_axis_name="c", subcore_axis_name="s",
        num_cores=NC, num_subcores=NS)

    @functools.partial(
        pl.kernel,
        out_type=jax.ShapeDtypeStruct((NW, 3, L), jnp.float32),
        mesh=mesh,
        scratch_types=[
            pltpu.VMEM((N * 7,), jnp.int32),
            pltpu.VMEM((N,), jnp.int32),
            pltpu.VMEM((C, TPW), jnp.int32),
            pltpu.VMEM((C, TPW), jnp.float32),
            pltpu.VMEM((3, L), jnp.float32),
            pltpu.SemaphoreType.DMA,
        ],
        compiler_params=pltpu.CompilerParams(needs_layout_passes=False),
    )
    def k(preds_hbm, tgt_hbm, out_hbm, tvals, flr, idxs, vals, part, sem):
        wid = lax.axis_index("s") * NC + lax.axis_index("c")
        b = wid // 2
        half = wid % 2

        pltpu.sync_copy(tgt_hbm.at[b], tvals)

        lane = lax.iota(jnp.int32, L)
        scale = jnp.float32(W / 80.0)
        for k4 in range(4):
            n16 = lane + (L * k4)
            t0 = plsc.load_gather(tvals, [n16 * 7])
            t1 = plsc.load_gather(tvals, [n16 * 7 + 1])
            gx = jnp.clip(t0.astype(jnp.float32) * scale,
                          0.0, float(W - 1)).astype(jnp.int32)
            gy = jnp.clip(t1.astype(jnp.float32) * scale,
                          0.0, float(H - 1)).astype(jnp.int32)
            flr[pl.ds(L * k4, L)] = gy * W + gx

        nA = half * TPW + lane
        nB = nA + L
        flA = plsc.load_gather(flr, [nA])
        flB = plsc.load_gather(flr, [nB])

        keepA = jnp.ones((L,), jnp.int32)
        keepB = jnp.ones((L,), jnp.int32)
        for j in range(N):
            fj = plsc.load_gather(flr, [jnp.full((L,), j, jnp.int32)])
            keepA = jnp.where((flA == fj) & (nA < j), 0, keepA)
            keepB = jnp.where((flB == fj) & (nB < j), 0, keepB)

        base = b * (C * HW)
        for c in range(C):
            idxs[c, pl.ds(0, L)] = base + c * HW + flA
            idxs[c, pl.ds(L, L)] = base + c * HW + flB

        copies = [
            pltpu.make_async_copy(preds_hbm.at[idxs.at[c]], vals.at[c], sem)
            for c in range(C)
        ]
        for cp in copies:
            cp.start()
        for cp in copies:
            cp.wait()

        acc_x = jnp.zeros((L,), jnp.float32)
        acc_s = jnp.zeros((L,), jnp.float32)
        acc_n = jnp.zeros((L,), jnp.float32)
        for r, (nloc, keep) in enumerate(((nA, keepA), (nB, keepB))):
            kf = keep.astype(jnp.float32)
            acc_n = acc_n + kf
            acc_x = acc_x + kf * vals[0, pl.ds(L * r, L)]
            for c in range(1, C):
                t = plsc.load_gather(
                    tvals, [nloc * 7 + (c - 1)]).astype(jnp.float32)
                d = vals[c, pl.ds(L * r, L)] - t
                ad = jnp.abs(d)
                sl1 = jnp.where(ad < 1.0, 0.5 * d * d, ad - 0.5)
                acc_s = acc_s + kf * sl1
        part[0, :] = acc_x
        part[1, :] = acc_s
        part[2, :] = acc_n
        pltpu.sync_copy(part, out_hbm.at[wid])

    return k(preds_flat, tgt_flat)


def kernel(preds, targets_list):
    sp_sum = _tc_softplus_sum(preds)[0, 0]
    parts = _sc_sparse_partials(
        preds.reshape(B * C * HW), targets_list.reshape(B, N * 7))
    sums = jnp.sum(parts, axis=(0, 2))
    num_objects = sums[2]
    cls_loss = (sp_sum - sums[0]) / jnp.float32(B * HW)
    reg_loss = sums[1] / (num_objects + 1e-06)
    total_loss = cls_loss + 2.0 * reg_loss
    return (total_loss, num_objects)


# speedup vs baseline: 12.3518x; 3.1410x over previous
import functools

import jax
import jax.numpy as jnp
from jax import lax
from jax.experimental import pallas as pl
from jax.experimental.pallas import tpu as pltpu
from jax.experimental.pallas import tpu_sc as plsc

B, C, H, W = 16, 8, 512, 512
HW = H * W
N = 64
NC, NS, L = 2, 16, 16
NW = NC * NS
TPW = (B * N) // NW


def _tc_softplus_sum(preds):
    def body(p_ref, o_ref):
        @pl.when(pl.program_id(0) == 0)
        def _():
            o_ref[0, 0] = 0.0

        x = p_ref[0, 0]
        o_ref[0, 0] += jnp.sum(
            jnp.maximum(x, 0.0) + jnp.log1p(jnp.exp(-jnp.abs(x))))

    return pl.pallas_call(
        body,
        grid=(B,),
        in_specs=[pl.BlockSpec((1, 1, H, W), lambda i: (i, 0, 0, 0))],
        out_specs=pl.BlockSpec(memory_space=pltpu.SMEM),
        out_shape=jax.ShapeDtypeStruct((1, 1), jnp.float32),
    )(preds)


def _sc_sparse_partials(preds_rows, tgt_flat):
    mesh = plsc.VectorSubcoreMesh(
        core_axis_name="c", subcore_axis_name="s",
        num_cores=NC, num_subcores=NS)

    @functools.partial(
        pl.kernel,
        out_type=jax.ShapeDtypeStruct((NW, 3, L), jnp.float32),
        mesh=mesh,
        scratch_types=[
            pltpu.VMEM((N * 7,), jnp.int32),
            pltpu.VMEM((N,), jnp.int32),
            pltpu.VMEM((TPW,), jnp.int32),
            pltpu.VMEM((TPW, W), jnp.float32),
            pltpu.VMEM((3, L), jnp.float32),
            pltpu.SemaphoreType.DMA,
        ],
        compiler_params=pltpu.CompilerParams(
            needs_layout_passes=False, use_tc_tiling_on_sc=True),
    )
    def k(preds_hbm, tgt_hbm, out_hbm, tvals, flr, ridx, rows, part, sem):
        wid = lax.axis_index("s") * NC + lax.axis_index("c")
        b = wid // 2
        half = wid % 2

        pltpu.sync_copy(tgt_hbm.at[pl.ds(b * (N * 7), N * 7)], tvals)

        lane = lax.iota(jnp.int32, L)
        scale = jnp.float32(W / 80.0)
        for k4 in range(4):
            n16 = lane + (L * k4)
            t0 = plsc.load_gather(tvals, [n16 * 7])
            t1 = plsc.load_gather(tvals, [n16 * 7 + 1])
            gx = jnp.clip(t0.astype(jnp.float32) * scale,
                          0.0, float(W - 1)).astype(jnp.int32)
            gy = jnp.clip(t1.astype(jnp.float32) * scale,
                          0.0, float(H - 1)).astype(jnp.int32)
            flr[pl.ds(L * k4, L)] = gy * W + gx

        nA = half * TPW + lane
        nB = nA + L
        flA = plsc.load_gather(flr, [nA])
        flB = plsc.load_gather(flr, [nB])

        keepA = jnp.ones((L,), jnp.int32)
        keepB = jnp.ones((L,), jnp.int32)
        for j in range(N):
            fj = plsc.load_gather(flr, [jnp.full((L,), j, jnp.int32)])
            keepA = jnp.where((flA == fj) & (nA < j), 0, keepA)
            keepB = jnp.where((flB == fj) & (nB < j), 0, keepB)
        kfA = keepA.astype(jnp.float32)
        kfB = keepB.astype(jnp.float32)

        gyA, gxA = lax.shift_right_logical(flA, 9), flA & (W - 1)
        gyB, gxB = lax.shift_right_logical(flB, 9), flB & (W - 1)

        acc_x = jnp.zeros((L,), jnp.float32)
        acc_s = jnp.zeros((L,), jnp.float32)
        rowbase = b * (C * H)
        for c in range(C):
            ridx[pl.ds(0, L)] = rowbase + c * H + gyA
            ridx[pl.ds(L, L)] = rowbase + c * H + gyB
            pltpu.async_copy(preds_hbm.at[ridx], rows, sem).wait()
            vA = plsc.load_gather(rows, [lane, gxA])
            vB = plsc.load_gather(rows, [lane + L, gxB])
            if c == 0:
                acc_x = kfA * vA + kfB * vB
            else:
                tA = plsc.load_gather(
                    tvals, [nA * 7 + (c - 1)]).astype(jnp.float32)
                tB = plsc.load_gather(
                    tvals, [nB * 7 + (c - 1)]).astype(jnp.float32)
                dA = vA - tA
                dB = vB - tB
                adA = jnp.abs(dA)
                adB = jnp.abs(dB)
                acc_s = acc_s + kfA * jnp.where(
                    adA < 1.0, 0.5 * dA * dA, adA - 0.5)
                acc_s = acc_s + kfB * jnp.where(
                    adB < 1.0, 0.5 * dB * dB, adB - 0.5)
        part[0, :] = acc_x
        part[1, :] = acc_s
        part[2, :] = kfA + kfB
        pltpu.sync_copy(part, out_hbm.at[wid])

    return k(preds_rows, tgt_flat)


def kernel(preds, targets_list):
    sp_sum = _tc_softplus_sum(preds)[0, 0]
    parts = _sc_sparse_partials(
        preds.reshape(B * C * H, W), targets_list.reshape(B * N * 7))
    sums = jnp.sum(parts, axis=(0, 2))
    num_objects = sums[2]
    cls_loss = (sp_sum - sums[0]) / jnp.float32(B * HW)
    reg_loss = sums[1] / (num_objects + 1e-06)
    total_loss = cls_loss + 2.0 * reg_loss
    return (total_loss, num_objects)


# speedup vs baseline: 12.6985x; 1.0281x over previous
import functools

import jax
import jax.numpy as jnp
from jax import lax
from jax.experimental import pallas as pl
from jax.experimental.pallas import tpu as pltpu
from jax.experimental.pallas import tpu_sc as plsc

B, C, H, W = 16, 8, 512, 512
HW = H * W
N = 64
NC, NS, L = 2, 16, 16
NW = NC * NS
TPW = (B * N) // NW
NBUF = 4
TCB = 2


def _tc_softplus_sum(preds):
    def body(p_ref, o_ref):
        @pl.when(pl.program_id(0) == 0)
        def _():
            o_ref[...] = jnp.zeros_like(o_ref)

        x = p_ref[...]
        sp = jnp.maximum(x, 0.0) + jnp.log1p(jnp.exp(-jnp.abs(x)))
        o_ref[0, :] += jnp.sum(sp, axis=(0, 1, 2))

    return pl.pallas_call(
        body,
        grid=(B // TCB,),
        in_specs=[pl.BlockSpec((TCB, 1, H, W), lambda i: (i, 0, 0, 0))],
        out_specs=pl.BlockSpec((1, W), lambda i: (0, 0)),
        out_shape=jax.ShapeDtypeStruct((1, W), jnp.float32),
    )(preds)


def _sc_sparse_partials(preds_rows, tgt_flat):
    mesh = plsc.VectorSubcoreMesh(
        core_axis_name="c", subcore_axis_name="s",
        num_cores=NC, num_subcores=NS)

    @functools.partial(
        pl.kernel,
        out_type=jax.ShapeDtypeStruct((NW, 3, L), jnp.float32),
        mesh=mesh,
        scratch_types=[
            pltpu.VMEM((N * 7,), jnp.int32),
            pltpu.VMEM((N,), jnp.int32),
            pltpu.VMEM((NBUF, TPW), jnp.int32),
            pltpu.VMEM((NBUF, TPW, W), jnp.float32),
            pltpu.VMEM((3, L), jnp.float32),
        ] + [pltpu.SemaphoreType.DMA] * NBUF,
        compiler_params=pltpu.CompilerParams(
            needs_layout_passes=False, use_tc_tiling_on_sc=True),
    )
    def k(preds_hbm, tgt_hbm, out_hbm, tvals, flr, ridx, rows, part, *sems):
        wid = lax.axis_index("s") * NC + lax.axis_index("c")
        b = wid // 2
        half = wid % 2

        pltpu.sync_copy(tgt_hbm.at[pl.ds(b * (N * 7), N * 7)], tvals)

        lane = lax.iota(jnp.int32, L)
        scale = jnp.float32(W / 80.0)
        for k4 in range(4):
            n16 = lane + (L * k4)
            t0 = plsc.load_gather(tvals, [n16 * 7])
            t1 = plsc.load_gather(tvals, [n16 * 7 + 1])
            gx = jnp.clip(t0.astype(jnp.float32) * scale,
                          0.0, float(W - 1)).astype(jnp.int32)
            gy = jnp.clip(t1.astype(jnp.float32) * scale,
                          0.0, float(H - 1)).astype(jnp.int32)
            flr[pl.ds(L * k4, L)] = gy * W + gx

        nA = half * TPW + lane
        nB = nA + L
        flA = plsc.load_gather(flr, [nA])
        flB = plsc.load_gather(flr, [nB])
        gyA, gxA = lax.shift_right_logical(flA, 9), flA & (W - 1)
        gyB, gxB = lax.shift_right_logical(flB, 9), flB & (W - 1)

        rowbase = b * (C * H)
        handles = [None] * NBUF

        def fill_issue(c):
            buf = c % NBUF
            ridx[buf, pl.ds(0, L)] = rowbase + c * H + gyA
            ridx[buf, pl.ds(L, L)] = rowbase + c * H + gyB
            handles[buf] = pltpu.async_copy(
                preds_hbm.at[ridx.at[buf]], rows.at[buf], sems[buf])

        for c in range(NBUF):
            fill_issue(c)

        keepA = jnp.ones((L,), jnp.int32)
        keepB = jnp.ones((L,), jnp.int32)
        for j in range(N):
            fj = plsc.load_gather(flr, [jnp.full((L,), j, jnp.int32)])
            keepA = jnp.where((flA == fj) & (nA < j), 0, keepA)
            keepB = jnp.where((flB == fj) & (nB < j), 0, keepB)
        kfA = keepA.astype(jnp.float32)
        kfB = keepB.astype(jnp.float32)

        acc_x = jnp.zeros((L,), jnp.float32)
        acc_s = jnp.zeros((L,), jnp.float32)
        for c in range(C):
            buf = c % NBUF
            handles[buf].wait()
            vA = plsc.load_gather(rows.at[buf], [lane, gxA])
            vB = plsc.load_gather(rows.at[buf], [lane + L, gxB])
            if c == 0:
                acc_x = kfA * vA + kfB * vB
            else:
                tA = plsc.load_gather(
                    tvals, [nA * 7 + (c - 1)]).astype(jnp.float32)
                tB = plsc.load_gather(
                    tvals, [nB * 7 + (c - 1)]).astype(jnp.float32)
                dA = vA - tA
                dB = vB - tB
                adA = jnp.abs(dA)
                adB = jnp.abs(dB)
                acc_s = acc_s + kfA * jnp.where(
                    adA < 1.0, 0.5 * dA * dA, adA - 0.5)
                acc_s = acc_s + kfB * jnp.where(
                    adB < 1.0, 0.5 * dB * dB, adB - 0.5)
            if c + NBUF < C:
                fill_issue(c + NBUF)
        part[0, :] = acc_x
        part[1, :] = acc_s
        part[2, :] = kfA + kfB
        pltpu.sync_copy(part, out_hbm.at[wid])

    return k(preds_rows, tgt_flat)


def kernel(preds, targets_list):
    parts = _sc_sparse_partials(
        preds.reshape(B * C * H, W), targets_list.reshape(B * N * 7))
    sp_part = _tc_softplus_sum(preds)
    sums = jnp.sum(parts, axis=(0, 2))
    num_objects = sums[2]
    cls_loss = (jnp.sum(sp_part) - sums[0]) / jnp.float32(B * HW)
    reg_loss = sums[1] / (num_objects + 1e-06)
    total_loss = cls_loss + 2.0 * reg_loss
    return (total_loss, num_objects)


# speedup vs baseline: 13.1176x; 1.0330x over previous
import functools

import jax
import jax.numpy as jnp
from jax import lax
from jax.experimental import pallas as pl
from jax.experimental.pallas import tpu as pltpu
from jax.experimental.pallas import tpu_sc as plsc

B, C, H, W = 16, 8, 512, 512
HW = H * W
N = 64
NC, NS, L = 2, 16, 16
NW = NC * NS
TPW = (B * N) // NW
NBUF = 4
TCB = 2


def _tc_softplus_sum(preds):
    def body(p_ref, o_ref):
        @pl.when(pl.program_id(0) == 0)
        def _():
            o_ref[...] = jnp.zeros_like(o_ref)

        x = p_ref[...]
        sp = jnp.maximum(x, 0.0) + jnp.log1p(jnp.exp(-jnp.abs(x)))
        o_ref[...] += jnp.sum(sp.reshape(TCB * (H // 8), 8, W), axis=0)

    return pl.pallas_call(
        body,
        grid=(B // TCB,),
        in_specs=[pl.BlockSpec((TCB, 1, H, W), lambda i: (i, 0, 0, 0))],
        out_specs=pl.BlockSpec((8, W), lambda i: (0, 0)),
        out_shape=jax.ShapeDtypeStruct((8, W), jnp.float32),
    )(preds)


def _sc_sparse_partials(preds_rows, tgt_flat):
    mesh = plsc.VectorSubcoreMesh(
        core_axis_name="c", subcore_axis_name="s",
        num_cores=NC, num_subcores=NS)

    @functools.partial(
        pl.kernel,
        out_type=jax.ShapeDtypeStruct((NW, L), jnp.float32),
        mesh=mesh,
        scratch_types=[
            pltpu.VMEM((N * 7,), jnp.int32),
            pltpu.VMEM((N,), jnp.int32),
            pltpu.VMEM((NBUF, TPW), jnp.int32),
            pltpu.VMEM((NBUF, TPW, W), jnp.float32),
            pltpu.VMEM((L,), jnp.float32),
        ] + [pltpu.SemaphoreType.DMA] * NBUF,
        compiler_params=pltpu.CompilerParams(
            needs_layout_passes=False, use_tc_tiling_on_sc=True),
    )
    def k(preds_hbm, tgt_hbm, out_hbm, tvals, flr, ridx, rows, part, *sems):
        wid = lax.axis_index("s") * NC + lax.axis_index("c")
        b = wid // 2
        half = wid % 2

        pltpu.sync_copy(tgt_hbm.at[pl.ds(b * (N * 7), N * 7)], tvals)

        lane = lax.iota(jnp.int32, L)
        scale = jnp.float32(W / 80.0)
        for k4 in range(4):
            n16 = lane + (L * k4)
            t0 = plsc.load_gather(tvals, [n16 * 7])
            t1 = plsc.load_gather(tvals, [n16 * 7 + 1])
            gx = jnp.clip(t0.astype(jnp.float32) * scale,
                          0.0, float(W - 1)).astype(jnp.int32)
            gy = jnp.clip(t1.astype(jnp.float32) * scale,
                          0.0, float(H - 1)).astype(jnp.int32)
            flr[pl.ds(L * k4, L)] = gy * W + gx

        nA = half * TPW + lane
        nB = nA + L
        flA = plsc.load_gather(flr, [nA])
        flB = plsc.load_gather(flr, [nB])
        gyA, gxA = lax.shift_right_logical(flA, 9), flA & (W - 1)
        gyB, gxB = lax.shift_right_logical(flB, 9), flB & (W - 1)

        rowbase = b * (C * H)
        handles = [None] * NBUF

        def fill_issue(c):
            buf = c % NBUF
            ridx[buf, pl.ds(0, L)] = rowbase + c * H + gyA
            ridx[buf, pl.ds(L, L)] = rowbase + c * H + gyB
            handles[buf] = pltpu.async_copy(
                preds_hbm.at[ridx.at[buf]], rows.at[buf], sems[buf])

        for c in range(NBUF):
            fill_issue(c)

        keepA = jnp.ones((L,), jnp.int32)
        keepB = jnp.ones((L,), jnp.int32)
        for j in range(N):
            fj = plsc.load_gather(flr, [jnp.full((L,), j, jnp.int32)])
            keepA = jnp.where((flA == fj) & (nA < j), 0, keepA)
            keepB = jnp.where((flB == fj) & (nB < j), 0, keepB)
        kfA = keepA.astype(jnp.float32)
        kfB = keepB.astype(jnp.float32)

        acc_x = jnp.zeros((L,), jnp.float32)
        acc_s = jnp.zeros((L,), jnp.float32)
        for c in range(C):
            buf = c % NBUF
            handles[buf].wait()
            vA = plsc.load_gather(rows.at[buf], [lane, gxA])
            vB = plsc.load_gather(rows.at[buf], [lane + L, gxB])
            if c == 0:
                acc_x = kfA * vA + kfB * vB
            else:
                tA = plsc.load_gather(
                    tvals, [nA * 7 + (c - 1)]).astype(jnp.float32)
                tB = plsc.load_gather(
                    tvals, [nB * 7 + (c - 1)]).astype(jnp.float32)
                dA = vA - tA
                dB = vB - tB
                adA = jnp.abs(dA)
                adB = jnp.abs(dB)
                acc_s = acc_s + kfA * jnp.where(
                    adA < 1.0, 0.5 * dA * dA, adA - 0.5)
                acc_s = acc_s + kfB * jnp.where(
                    adB < 1.0, 0.5 * dB * dB, adB - 0.5)
            if c + NBUF < C:
                fill_issue(c + NBUF)
        sx = jnp.sum(acc_x)
        ss = jnp.sum(acc_s)
        sn = jnp.sum(kfA + kfB)
        part[...] = jnp.where(
            lane == 0, sx,
            jnp.where(lane == 1, ss,
                      jnp.where(lane == 2, sn, jnp.float32(0.0))))
        pltpu.sync_copy(part, out_hbm.at[wid])

    return k(preds_rows, tgt_flat)


def kernel(preds, targets_list):
    parts = _sc_sparse_partials(
        preds.reshape(B * C * H, W), targets_list.reshape(B * N * 7))
    sp_part = _tc_softplus_sum(preds)
    sums = jnp.sum(parts, axis=0)
    num_objects = sums[2]
    cls_loss = (jnp.sum(sp_part) - sums[0]) / jnp.float32(B * HW)
    reg_loss = sums[1] / (num_objects + 1e-06)
    total_loss = cls_loss + 2.0 * reg_loss
    return (total_loss, num_objects)


# speedup vs baseline: 13.4774x; 1.0274x over previous
import functools

import jax
import jax.numpy as jnp
from jax import lax
from jax.experimental import pallas as pl
from jax.experimental.pallas import tpu as pltpu
from jax.experimental.pallas import tpu_sc as plsc

B, C, H, W = 16, 8, 512, 512
HW = H * W
N = 64
NC, NS, L = 2, 16, 16
NW = NC * NS
TPW = (B * N) // NW
NBUF = 4
TCB = 2


def _tc_softplus_sum(preds):
    NSTR = 4

    def body(*refs):
        p_refs, o_ref = refs[:NSTR], refs[NSTR]

        @pl.when(pl.program_id(0) == 0)
        def _():
            o_ref[...] = jnp.zeros_like(o_ref)

        acc = o_ref[...]
        for p_ref in p_refs:
            x = p_ref[...]
            sp = jnp.maximum(x, 0.0) + jnp.log1p(jnp.exp(-jnp.abs(x)))
            acc = acc + jnp.sum(sp.reshape(H // 8, 8, W), axis=0)
        o_ref[...] = acc

    steps = B // NSTR
    return pl.pallas_call(
        body,
        grid=(steps,),
        in_specs=[
            pl.BlockSpec((1, 1, H, W),
                         lambda i, s=s: (s * steps + i, 0, 0, 0))
            for s in range(NSTR)
        ],
        out_specs=pl.BlockSpec((8, W), lambda i: (0, 0)),
        out_shape=jax.ShapeDtypeStruct((8, W), jnp.float32),
    )(*([preds] * NSTR))


def _sc_sparse_partials(preds_rows, tgt_flat):
    mesh = plsc.VectorSubcoreMesh(
        core_axis_name="c", subcore_axis_name="s",
        num_cores=NC, num_subcores=NS)

    @functools.partial(
        pl.kernel,
        out_type=jax.ShapeDtypeStruct((NW, L), jnp.float32),
        mesh=mesh,
        scratch_types=[
            pltpu.VMEM((N * 7,), jnp.int32),
            pltpu.VMEM((N,), jnp.int32),
            pltpu.VMEM((NBUF, TPW), jnp.int32),
            pltpu.VMEM((NBUF, TPW, W), jnp.float32),
            pltpu.VMEM((L,), jnp.float32),
        ] + [pltpu.SemaphoreType.DMA] * NBUF,
        compiler_params=pltpu.CompilerParams(
            needs_layout_passes=False, use_tc_tiling_on_sc=True),
    )
    def k(preds_hbm, tgt_hbm, out_hbm, tvals, flr, ridx, rows, part, *sems):
        wid = lax.axis_index("s") * NC + lax.axis_index("c")
        b = wid // 2
        half = wid % 2

        pltpu.sync_copy(tgt_hbm.at[pl.ds(b * (N * 7), N * 7)], tvals)

        lane = lax.iota(jnp.int32, L)
        scale = jnp.float32(W / 80.0)
        for k4 in range(4):
            n16 = lane + (L * k4)
            t0 = plsc.load_gather(tvals, [n16 * 7])
            t1 = plsc.load_gather(tvals, [n16 * 7 + 1])
            gx = jnp.clip(t0.astype(jnp.float32) * scale,
                          0.0, float(W - 1)).astype(jnp.int32)
            gy = jnp.clip(t1.astype(jnp.float32) * scale,
                          0.0, float(H - 1)).astype(jnp.int32)
            flr[pl.ds(L * k4, L)] = gy * W + gx

        nA = half * TPW + lane
        nB = nA + L
        flA = plsc.load_gather(flr, [nA])
        flB = plsc.load_gather(flr, [nB])
        gyA, gxA = lax.shift_right_logical(flA, 9), flA & (W - 1)
        gyB, gxB = lax.shift_right_logical(flB, 9), flB & (W - 1)

        rowbase = b * (C * H)
        handles = [None] * NBUF

        def fill_issue(c):
            buf = c % NBUF
            ridx[buf, pl.ds(0, L)] = rowbase + c * H + gyA
            ridx[buf, pl.ds(L, L)] = rowbase + c * H + gyB
            handles[buf] = pltpu.async_copy(
                preds_hbm.at[ridx.at[buf]], rows.at[buf], sems[buf])

        for c in range(NBUF):
            fill_issue(c)

        keepA = jnp.ones((L,), jnp.int32)
        keepB = jnp.ones((L,), jnp.int32)
        for j in range(N):
            fj = plsc.load_gather(flr, [jnp.full((L,), j, jnp.int32)])
            keepA = jnp.where((flA == fj) & (nA < j), 0, keepA)
            keepB = jnp.where((flB == fj) & (nB < j), 0, keepB)
        kfA = keepA.astype(jnp.float32)
        kfB = keepB.astype(jnp.float32)

        acc_x = jnp.zeros((L,), jnp.float32)
        acc_s = jnp.zeros((L,), jnp.float32)
        for c in range(C):
            buf = c % NBUF
            handles[buf].wait()
            vA = plsc.load_gather(rows.at[buf], [lane, gxA])
            vB = plsc.load_gather(rows.at[buf], [lane + L, gxB])
            if c == 0:
                acc_x = kfA * vA + kfB * vB
            else:
                tA = plsc.load_gather(
                    tvals, [nA * 7 + (c - 1)]).astype(jnp.float32)
                tB = plsc.load_gather(
                    tvals, [nB * 7 + (c - 1)]).astype(jnp.float32)
                dA = vA - tA
                dB = vB - tB
                adA = jnp.abs(dA)
                adB = jnp.abs(dB)
                acc_s = acc_s + kfA * jnp.where(
                    adA < 1.0, 0.5 * dA * dA, adA - 0.5)
                acc_s = acc_s + kfB * jnp.where(
                    adB < 1.0, 0.5 * dB * dB, adB - 0.5)
            if c + NBUF < C:
                fill_issue(c + NBUF)
        sx = jnp.sum(acc_x)
        ss = jnp.sum(acc_s)
        sn = jnp.sum(kfA + kfB)
        part[...] = jnp.where(
            lane == 0, sx,
            jnp.where(lane == 1, ss,
                      jnp.where(lane == 2, sn, jnp.float32(0.0))))
        pltpu.sync_copy(part, out_hbm.at[wid])

    return k(preds_rows, tgt_flat)


def kernel(preds, targets_list):
    parts = _sc_sparse_partials(
        preds.reshape(B * C * H, W), targets_list.reshape(B * N * 7))
    sp_part = _tc_softplus_sum(preds)
    sums = jnp.sum(parts, axis=0)
    num_objects = sums[2]
    cls_loss = (jnp.sum(sp_part) - sums[0]) / jnp.float32(B * HW)
    reg_loss = sums[1] / (num_objects + 1e-06)
    total_loss = cls_loss + 2.0 * reg_loss
    return (total_loss, num_objects)


# speedup vs baseline: 13.8447x; 1.0273x over previous
import functools

import jax
import jax.numpy as jnp
from jax import lax
from jax.experimental import pallas as pl
from jax.experimental.pallas import tpu as pltpu
from jax.experimental.pallas import tpu_sc as plsc

B, C, H, W = 16, 8, 512, 512
HW = H * W
N = 64
NC, NS, L = 2, 16, 16
NW = NC * NS
TPW = (B * N) // NW
NBUF = 4
TCB = 2


def _tc_softplus_sum(preds):
    NSTR = 4
    steps = B // NSTR

    def body(*refs):
        p_refs, o_ref, acc_ref = refs[:NSTR], refs[NSTR], refs[NSTR + 1]

        @pl.when(pl.program_id(0) == 0)
        def _():
            acc_ref[...] = jnp.zeros_like(acc_ref)

        acc = acc_ref[...]
        for p_ref in p_refs:
            x = p_ref[...]
            sp = jnp.maximum(x, 0.0) + jnp.log1p(jnp.exp(-jnp.abs(x)))
            acc = acc + jnp.sum(sp.reshape(H // 8, 8, W), axis=0)
        acc_ref[...] = acc

        @pl.when(pl.program_id(0) == steps - 1)
        def _():
            o_ref[0, 0] = jnp.sum(acc)

    return pl.pallas_call(
        body,
        grid=(steps,),
        in_specs=[
            pl.BlockSpec((1, 1, H, W),
                         lambda i, s=s: (s * steps + i, 0, 0, 0))
            for s in range(NSTR)
        ],
        out_specs=pl.BlockSpec(memory_space=pltpu.SMEM),
        out_shape=jax.ShapeDtypeStruct((1, 1), jnp.float32),
        scratch_shapes=[pltpu.VMEM((8, W), jnp.float32)],
    )(*([preds] * NSTR))


def _sc_sparse_partials(preds_rows, tgt_flat):
    mesh = plsc.VectorSubcoreMesh(
        core_axis_name="c", subcore_axis_name="s",
        num_cores=NC, num_subcores=NS)

    @functools.partial(
        pl.kernel,
        out_type=jax.ShapeDtypeStruct((NW, L), jnp.float32),
        mesh=mesh,
        scratch_types=[
            pltpu.VMEM((N * 7,), jnp.int32),
            pltpu.VMEM((N,), jnp.int32),
            pltpu.VMEM((NBUF, TPW), jnp.int32),
            pltpu.VMEM((NBUF, TPW, W), jnp.float32),
            pltpu.VMEM((L,), jnp.float32),
        ] + [pltpu.SemaphoreType.DMA] * NBUF,
        compiler_params=pltpu.CompilerParams(
            needs_layout_passes=False, use_tc_tiling_on_sc=True),
    )
    def k(preds_hbm, tgt_hbm, out_hbm, tvals, flr, ridx, rows, part, *sems):
        wid = lax.axis_index("s") * NC + lax.axis_index("c")
        b = wid // 2
        half = wid % 2

        pltpu.sync_copy(tgt_hbm.at[pl.ds(b * (N * 7), N * 7)], tvals)

        lane = lax.iota(jnp.int32, L)
        scale = jnp.float32(W / 80.0)
        for k4 in range(4):
            n16 = lane + (L * k4)
            t0 = plsc.load_gather(tvals, [n16 * 7])
            t1 = plsc.load_gather(tvals, [n16 * 7 + 1])
            gx = jnp.clip(t0.astype(jnp.float32) * scale,
                          0.0, float(W - 1)).astype(jnp.int32)
            gy = jnp.clip(t1.astype(jnp.float32) * scale,
                          0.0, float(H - 1)).astype(jnp.int32)
            flr[pl.ds(L * k4, L)] = gy * W + gx

        nA = half * TPW + lane
        nB = nA + L
        flA = plsc.load_gather(flr, [nA])
        flB = plsc.load_gather(flr, [nB])
        gyA, gxA = lax.shift_right_logical(flA, 9), flA & (W - 1)
        gyB, gxB = lax.shift_right_logical(flB, 9), flB & (W - 1)

        rowbase = b * (C * H)
        handles = [None] * NBUF

        def fill_issue(c):
            buf = c % NBUF
            ridx[buf, pl.ds(0, L)] = rowbase + c * H + gyA
            ridx[buf, pl.ds(L, L)] = rowbase + c * H + gyB
            handles[buf] = pltpu.async_copy(
                preds_hbm.at[ridx.at[buf]], rows.at[buf], sems[buf])

        for c in range(NBUF):
            fill_issue(c)

        def dedup_body(j, carry):
            kA, kB = carry
            fj = plsc.load_gather(flr, [jnp.broadcast_to(j, (L,))])
            kA = jnp.where((flA == fj) & (nA < j), 0, kA)
            kB = jnp.where((flB == fj) & (nB < j), 0, kB)
            return kA, kB

        keepA, keepB = lax.fori_loop(
            0, N, dedup_body,
            (jnp.ones((L,), jnp.int32), jnp.ones((L,), jnp.int32)))
        kfA = keepA.astype(jnp.float32)
        kfB = keepB.astype(jnp.float32)

        acc_x = jnp.zeros((L,), jnp.float32)
        acc_s = jnp.zeros((L,), jnp.float32)
        for c in range(C):
            buf = c % NBUF
            handles[buf].wait()
            vA = plsc.load_gather(rows.at[buf], [lane, gxA])
            vB = plsc.load_gather(rows.at[buf], [lane + L, gxB])
            if c == 0:
                acc_x = kfA * vA + kfB * vB
            else:
                tA = plsc.load_gather(
                    tvals, [nA * 7 + (c - 1)]).astype(jnp.float32)
                tB = plsc.load_gather(
                    tvals, [nB * 7 + (c - 1)]).astype(jnp.float32)
                dA = vA - tA
                dB = vB - tB
                adA = jnp.abs(dA)
                adB = jnp.abs(dB)
                acc_s = acc_s + kfA * jnp.where(
                    adA < 1.0, 0.5 * dA * dA, adA - 0.5)
                acc_s = acc_s + kfB * jnp.where(
                    adB < 1.0, 0.5 * dB * dB, adB - 0.5)
            if c + NBUF < C:
                fill_issue(c + NBUF)
        sx = jnp.sum(acc_x)
        ss = jnp.sum(acc_s)
        sn = jnp.sum(kfA + kfB)
        part[...] = jnp.where(
            lane == 0, sx,
            jnp.where(lane == 1, ss,
                      jnp.where(lane == 2, sn, jnp.float32(0.0))))
        pltpu.sync_copy(part, out_hbm.at[wid])

    return k(preds_rows, tgt_flat)


def kernel(preds, targets_list):
    parts = _sc_sparse_partials(
        preds.reshape(B * C * H, W), targets_list.reshape(B * N * 7))
    sp_sum = _tc_softplus_sum(preds)[0, 0]
    sums = jnp.sum(parts, axis=0)
    num_objects = sums[2]
    cls_loss = (sp_sum - sums[0]) / jnp.float32(B * HW)
    reg_loss = sums[1] / (num_objects + 1e-06)
    total_loss = cls_loss + 2.0 * reg_loss
    return (total_loss, num_objects)


# speedup vs baseline: 13.9653x; 1.0087x over previous
import functools

import jax
import jax.numpy as jnp
from jax import lax
from jax.experimental import pallas as pl
from jax.experimental.pallas import tpu as pltpu
from jax.experimental.pallas import tpu_sc as plsc

B, C, H, W = 16, 8, 512, 512
HW = H * W
N = 64
NC, NS, L = 2, 16, 16
NW = NC * NS
TPW = (B * N) // NW
NBUF = 4
TCB = 2


def _tc_softplus_sum(preds):
    NSTR = 4
    steps = B // NSTR

    def body(*refs):
        p_refs, o_ref, acc_ref = refs[:NSTR], refs[NSTR], refs[NSTR + 1]

        @pl.when(pl.program_id(0) == 0)
        def _():
            acc_ref[...] = jnp.zeros_like(acc_ref)

        acc = acc_ref[...]
        for p_ref in p_refs:
            x = p_ref[...]
            sp = jnp.maximum(x, 0.0) + jnp.log1p(jnp.exp(-jnp.abs(x)))
            acc = acc + jnp.sum(sp.reshape(H // 8, 8, W), axis=0)
        acc_ref[...] = acc

        @pl.when(pl.program_id(0) == steps - 1)
        def _():
            o_ref[0, 0] = jnp.sum(acc)

    return pl.pallas_call(
        body,
        grid=(steps,),
        in_specs=[
            pl.BlockSpec((1, 1, H, W),
                         lambda i, s=s: (s * steps + i, 0, 0, 0))
            for s in range(NSTR)
        ],
        out_specs=pl.BlockSpec(memory_space=pltpu.SMEM),
        out_shape=jax.ShapeDtypeStruct((1, 1), jnp.float32),
        scratch_shapes=[pltpu.VMEM((8, W), jnp.float32)],
    )(*([preds] * NSTR))


def _sc_sparse_partials(preds_rows, tgt_flat):
    mesh = plsc.VectorSubcoreMesh(
        core_axis_name="c", subcore_axis_name="s",
        num_cores=NC, num_subcores=NS)

    @functools.partial(
        pl.kernel,
        out_type=jax.ShapeDtypeStruct((NW, L), jnp.float32),
        mesh=mesh,
        scratch_types=[
            pltpu.VMEM((N, 7), jnp.int32),
            pltpu.VMEM((N,), jnp.int32),
            pltpu.VMEM((NBUF, TPW), jnp.int32),
            pltpu.VMEM((NBUF, TPW, W), jnp.float32),
            pltpu.VMEM((L,), jnp.float32),
        ] + [pltpu.SemaphoreType.DMA] * NBUF,
        compiler_params=pltpu.CompilerParams(
            needs_layout_passes=False, use_tc_tiling_on_sc=True),
    )
    def k(preds_hbm, tgt_hbm, out_hbm, tvals, flr, ridx, rows, part, *sems):
        wid = lax.axis_index("s") * NC + lax.axis_index("c")
        b = wid // 2
        half = wid % 2

        pltpu.sync_copy(tgt_hbm.at[pl.ds(b * N, N)], tvals)

        lane = lax.iota(jnp.int32, L)
        zero = jnp.zeros((L,), jnp.int32)
        scale = jnp.float32(W / 80.0)
        for k4 in range(4):
            n16 = lane + (L * k4)
            t0 = plsc.load_gather(tvals, [n16, zero])
            t1 = plsc.load_gather(tvals, [n16, zero + 1])
            gx = jnp.clip(t0.astype(jnp.float32) * scale,
                          0.0, float(W - 1)).astype(jnp.int32)
            gy = jnp.clip(t1.astype(jnp.float32) * scale,
                          0.0, float(H - 1)).astype(jnp.int32)
            flr[pl.ds(L * k4, L)] = gy * W + gx

        nA = half * TPW + lane
        nB = nA + L
        flA = plsc.load_gather(flr, [nA])
        flB = plsc.load_gather(flr, [nB])
        gyA, gxA = lax.shift_right_logical(flA, 9), flA & (W - 1)
        gyB, gxB = lax.shift_right_logical(flB, 9), flB & (W - 1)

        rowbase = b * (C * H)
        handles = [None] * NBUF

        def fill_issue(c):
            buf = c % NBUF
            ridx[buf, pl.ds(0, L)] = rowbase + c * H + gyA
            ridx[buf, pl.ds(L, L)] = rowbase + c * H + gyB
            handles[buf] = pltpu.async_copy(
                preds_hbm.at[ridx.at[buf]], rows.at[buf], sems[buf])

        for c in range(NBUF):
            fill_issue(c)

        def dedup_body(j, carry):
            kA, kB = carry
            fj = plsc.load_gather(flr, [jnp.broadcast_to(j, (L,))])
            kA = jnp.where((flA == fj) & (nA < j), 0, kA)
            kB = jnp.where((flB == fj) & (nB < j), 0, kB)
            return kA, kB

        keepA, keepB = lax.fori_loop(
            0, N, dedup_body,
            (jnp.ones((L,), jnp.int32), jnp.ones((L,), jnp.int32)))
        kfA = keepA.astype(jnp.float32)
        kfB = keepB.astype(jnp.float32)

        acc_x = jnp.zeros((L,), jnp.float32)
        acc_s = jnp.zeros((L,), jnp.float32)
        for c in range(C):
            buf = c % NBUF
            handles[buf].wait()
            vA = plsc.load_gather(rows.at[buf], [lane, gxA])
            vB = plsc.load_gather(rows.at[buf], [lane + L, gxB])
            if c == 0:
                acc_x = kfA * vA + kfB * vB
            else:
                tA = plsc.load_gather(
                    tvals, [nA, zero + (c - 1)]).astype(jnp.float32)
                tB = plsc.load_gather(
                    tvals, [nB, zero + (c - 1)]).astype(jnp.float32)
                dA = vA - tA
                dB = vB - tB
                adA = jnp.abs(dA)
                adB = jnp.abs(dB)
                acc_s = acc_s + kfA * jnp.where(
                    adA < 1.0, 0.5 * dA * dA, adA - 0.5)
                acc_s = acc_s + kfB * jnp.where(
                    adB < 1.0, 0.5 * dB * dB, adB - 0.5)
            if c + NBUF < C:
                fill_issue(c + NBUF)
        sx = jnp.sum(acc_x)
        ss = jnp.sum(acc_s)
        sn = jnp.sum(kfA + kfB)
        part[...] = jnp.where(
            lane == 0, sx,
            jnp.where(lane == 1, ss,
                      jnp.where(lane == 2, sn, jnp.float32(0.0))))
        pltpu.sync_copy(part, out_hbm.at[wid])

    return k(preds_rows, tgt_flat)


def kernel(preds, targets_list):
    parts = _sc_sparse_partials(
        preds.reshape(B * C * H, W), targets_list.reshape(B * N, 7))
    sp_sum = _tc_softplus_sum(preds)[0, 0]
    sums = jnp.sum(parts, axis=0)
    num_objects = sums[2]
    cls_loss = (sp_sum - sums[0]) / jnp.float32(B * HW)
    reg_loss = sums[1] / (num_objects + 1e-06)
    total_loss = cls_loss + 2.0 * reg_loss
    return (total_loss, num_objects)


# speedup vs baseline: 13.9707x; 1.0004x over previous
import functools

import jax
import jax.numpy as jnp
from jax import lax
from jax.experimental import pallas as pl
from jax.experimental.pallas import tpu as pltpu
from jax.experimental.pallas import tpu_sc as plsc

B, C, H, W = 16, 8, 512, 512
HW = H * W
N = 64
NC, NS, L = 2, 16, 16
NW = NC * NS
TPW = (B * N) // NW
NBUF = 4
TCB = 2


def _tc_softplus_sum(preds):
    NSTR = 4
    steps = B // NSTR

    def body(*refs):
        p_refs, o_ref, acc_ref = refs[:NSTR], refs[NSTR], refs[NSTR + 1]

        @pl.when(pl.program_id(0) == 0)
        def _():
            acc_ref[...] = jnp.zeros_like(acc_ref)

        acc = acc_ref[...]
        for p_ref in p_refs:
            x = p_ref[...]
            sp = jnp.maximum(x, 0.0) + jnp.log1p(jnp.exp(-jnp.abs(x)))
            acc = acc + jnp.sum(sp.reshape(H // 8, 8, W), axis=0)
        acc_ref[...] = acc

        @pl.when(pl.program_id(0) == steps - 1)
        def _():
            o_ref[0, 0] = jnp.sum(acc)

    return pl.pallas_call(
        body,
        grid=(steps,),
        in_specs=[
            pl.BlockSpec((1, 1, H, W),
                         lambda i, s=s: (s * steps + i, 0, 0, 0))
            for s in range(NSTR)
        ],
        out_specs=pl.BlockSpec(memory_space=pltpu.SMEM),
        out_shape=jax.ShapeDtypeStruct((1, 1), jnp.float32),
        scratch_shapes=[pltpu.VMEM((8, W), jnp.float32)],
    )(*([preds] * NSTR))


def _sc_sparse_partials(preds_rows, tgt_rows):
    mesh = plsc.VectorSubcoreMesh(
        core_axis_name="c", subcore_axis_name="s",
        num_cores=NC, num_subcores=NS)

    @functools.partial(
        pl.kernel,
        out_type=jax.ShapeDtypeStruct((NW, L), jnp.float32),
        mesh=mesh,
        scratch_types=[
            pltpu.VMEM((N, 7), jnp.int32),
            pltpu.VMEM((N,), jnp.int32),
            pltpu.VMEM((NBUF, TPW), jnp.int32),
            pltpu.VMEM((NBUF, TPW, W), jnp.float32),
            pltpu.VMEM((L,), jnp.float32),
        ] + [pltpu.SemaphoreType.DMA] * NBUF,
        compiler_params=pltpu.CompilerParams(
            needs_layout_passes=False, use_tc_tiling_on_sc=True),
    )
    def k(preds_hbm, tgt_hbm, out_hbm, tvals, flr, ridx, rows, part, *sems):
        wid = lax.axis_index("s") * NC + lax.axis_index("c")
        b = wid // 2
        half = wid % 2

        pltpu.sync_copy(tgt_hbm.at[pl.ds(b * N, N)], tvals)

        lane = lax.iota(jnp.int32, L)
        zero = jnp.zeros((L,), jnp.int32)
        scale = jnp.float32(W / 80.0)
        for k4 in range(4):
            n16 = lane + (L * k4)
            t0 = plsc.load_gather(tvals, [n16, zero])
            t1 = plsc.load_gather(tvals, [n16, zero + 1])
            gx = jnp.clip(t0.astype(jnp.float32) * scale,
                          0.0, float(W - 1)).astype(jnp.int32)
            gy = jnp.clip(t1.astype(jnp.float32) * scale,
                          0.0, float(H - 1)).astype(jnp.int32)
            flr[pl.ds(L * k4, L)] = gy * W + gx

        nA = half * TPW + lane
        nB = nA + L
        flA = plsc.load_gather(flr, [nA])
        flB = plsc.load_gather(flr, [nB])
        gyA, gxA = lax.shift_right_logical(flA, 9), flA & (W - 1)
        gyB, gxB = lax.shift_right_logical(flB, 9), flB & (W - 1)

        rowbase = b * (C * H)
        handles = [None] * NBUF

        def fill_issue(c):
            buf = c % NBUF
            ridx[buf, pl.ds(0, L)] = rowbase + c * H + gyA
            ridx[buf, pl.ds(L, L)] = rowbase + c * H + gyB
            handles[buf] = pltpu.async_copy(
                preds_hbm.at[ridx.at[buf]], rows.at[buf], sems[buf])

        for c in range(NBUF):
            fill_issue(c)

        def dedup_body(j, carry):
            kA, kB = carry
            fj = plsc.load_gather(flr, [jnp.broadcast_to(j, (L,))])
            kA = jnp.where((flA == fj) & (nA < j), 0, kA)
            kB = jnp.where((flB == fj) & (nB < j), 0, kB)
            return kA, kB

        keepA, keepB = lax.fori_loop(
            0, N, dedup_body,
            (jnp.ones((L,), jnp.int32), jnp.ones((L,), jnp.int32)))
        kfA = keepA.astype(jnp.float32)
        kfB = keepB.astype(jnp.float32)

        acc_x = jnp.zeros((L,), jnp.float32)
        acc_s = jnp.zeros((L,), jnp.float32)
        for c in range(C):
            buf = c % NBUF
            handles[buf].wait()
            vA = plsc.load_gather(rows.at[buf], [lane, gxA])
            vB = plsc.load_gather(rows.at[buf], [lane + L, gxB])
            if c == 0:
                acc_x = kfA * vA + kfB * vB
            else:
                tA = plsc.load_gather(
                    tvals, [nA, zero + (c - 1)]).astype(jnp.float32)
                tB = plsc.load_gather(
                    tvals, [nB, zero + (c - 1)]).astype(jnp.float32)
                dA = vA - tA
                dB = vB - tB
                adA = jnp.abs(dA)
                adB = jnp.abs(dB)
                acc_s = acc_s + kfA * jnp.where(
                    adA < 1.0, 0.5 * dA * dA, adA - 0.5)
                acc_s = acc_s + kfB * jnp.where(
                    adB < 1.0, 0.5 * dB * dB, adB - 0.5)
            if c + NBUF < C:
                fill_issue(c + NBUF)
        sx = jnp.sum(acc_x)
        ss = jnp.sum(acc_s)
        sn = jnp.sum(kfA + kfB)
        part[...] = jnp.where(
            lane == 0, sx,
            jnp.where(lane == 1, ss,
                      jnp.where(lane == 2, sn, jnp.float32(0.0))))
        pltpu.sync_copy(part, out_hbm.at[wid])

    return k(preds_rows, tgt_rows)


def kernel(preds, targets_list):
    parts = _sc_sparse_partials(
        preds.reshape(B * C * H, W), targets_list.reshape(B * N, 7))
    sp_sum = _tc_softplus_sum(preds)[0, 0]
    sums = jnp.sum(parts, axis=0)
    num_objects = sums[2]
    cls_loss = (sp_sum - sums[0]) / jnp.float32(B * HW)
    reg_loss = sums[1] / (num_objects + 1e-06)
    total_loss = cls_loss + 2.0 * reg_loss
    return (total_loss, num_objects)
